# baseline (speedup 1.0000x reference)
import functools

import jax
import jax.numpy as jnp
from jax.experimental import pallas as pl
from jax.experimental.pallas import tpu as pltpu

K = 16
K2 = 2
BF = jnp.bfloat16
F32 = jnp.float32


def _dot(a, b, prec=F32):
    return jax.lax.dot_general(a, b, (((1,), (0,)), ((), ())),
                               preferred_element_type=prec)


def _silu(x):
    return x * (0.5 + 0.5 * jnp.tanh(0.5 * x))


def _geo_body(BNg, px_r, py_r, lenT_o, dirxT_o, diryT_o, geoP_o):
    px = jnp.concatenate([px_r[...], px_r[:, :K + 2]], axis=1)
    py = jnp.concatenate([py_r[...], py_r[:, :K + 2]], axis=1)
    M = BNg + 1
    pjx = jnp.concatenate([px[:, k + 1:k + 1 + M] for k in range(K)], axis=0)
    pjy = jnp.concatenate([py[:, k + 1:k + 1 + M] for k in range(K)], axis=0)
    pix = jnp.broadcast_to(px[:, :M], (K, M))
    piy = jnp.broadcast_to(py[:, :M], (K, M))
    relx = pjx - pix
    rely = pjy - piy
    ss = relx * relx + rely * rely
    inv = jax.lax.rsqrt(ss + 1e-12)
    lng = ss * inv
    dirx = relx * inv
    diry = rely * inv
    mpx = 0.5 * (pix + pjx)
    mpy = 0.5 * (piy + pjy)

    def s1(X):
        return jnp.concatenate([X[1:, :BNg], X[0:1, 1:BNg + 1]], axis=0)

    def s2(X):
        return jnp.concatenate([X[2:, :BNg], X[0:2, 1:BNg + 1]], axis=0)

    dx1 = s1(mpx) - mpx[:, :BNg]
    dy1 = s1(mpy) - mpy[:, :BNg]
    dx2 = s2(mpx) - mpx[:, :BNg]
    dy2 = s2(mpy) - mpy[:, :BNg]
    d1 = jnp.sqrt(dx1 * dx1 + dy1 * dy1 + 1e-12)
    d2 = jnp.sqrt(dx2 * dx2 + dy2 * dy2 + 1e-12)
    c1 = s1(dirx) * dirx[:, :BNg] + s1(diry) * diry[:, :BNg]
    c2 = s2(dirx) * dirx[:, :BNg] + s2(diry) * diry[:, :BNg]

    tr = lambda x: jnp.transpose(x, (1, 0)).astype(BF)
    lenT_o[...] = tr(lng[:, :BNg])
    dirxT_o[...] = tr(dirx[:, :BNg])
    diryT_o[...] = tr(diry[:, :BNg])
    geoP_o[...] = jnp.concatenate([tr(d1), tr(c1), tr(d2), tr(c2)], axis=1)


def _stage1_body(F, BN, BE, SD, VD,
                 fA, fB, lenT, dxyT, W1a, W1b, W1c, WlenB, b1, W2blk, b2,
                 DB2, h_o, vc_o, hh_o, vch_o):
    fext = jnp.concatenate([fA[...], fB[...]], axis=0).astype(BF)
    fi = jnp.broadcast_to(fext[:BN][:, None, :], (BN, K, F)).reshape(BE, F)
    fj = jnp.concatenate(
        [fext[k + 1:k + 1 + BN][:, None, :] for k in range(K)],
        axis=1).reshape(BE, F)
    diff = jnp.abs(fi - fj)

    lenadd = _dot(lenT[...], WlenB[...]).astype(BF).reshape(BE, 128)
    pre = (_dot(fi, W1a[...]) + _dot(fj, W1b[...]) + _dot(diff, W1c[...])
           + lenadd + b1[...])
    u = _silu(pre).astype(BF)
    r = _dot(u, W2blk[...]) + b2[...]
    h = r[:, :SD]

    dxy = _dot(dxyT[...], DB2[...]).astype(BF).reshape(BE, 128)
    vc = r[:, SD:] * dxy[:, :2 * VD]
    h_o[...] = h
    vc_o[...] = vc
    hh_o[...] = h[:2][None]
    vch_o[...] = vc[:2][None]


def _layer_body(BE, SD, VD, HID, final,
                h_r, hh_r, vc_r, vch_r, geoP_r,
                A, Bm, leb1, leW2, leb2x2, WgeoB, W1t, W1bot, b1u,
                lnW2, lnb2, lvW2d, lvb2d,
                h_o, v_o, *rest):
    h = h_r[...]
    hb = h.astype(BF)
    hextb = jnp.concatenate([hb, hh_r[0].astype(BF)], axis=0)
    hA = _dot(hextb, A[...])
    s1 = hA[1:BE + 1]
    s2 = hA[2:BE + 2]
    hB = _dot(hb, Bm[...]) + leb1[...]
    g = _dot(geoP_r[...], WgeoB[...]).astype(BF).reshape(BE, 128)
    m1 = _silu(s1 + hB + g[:, :HID]).astype(BF)
    m2 = _silu(s2 + hB + g[:, HID:]).astype(BF)
    agg = _dot(m1, leW2[...]) + _dot(m2, leW2[...]) + leb2x2[...]

    u = _dot(hb, W1t[...]) + _dot(agg.astype(BF), W1bot[...]) + b1u[...]
    us = _silu(u).astype(BF)
    t = _dot(us[:, :HID], lnW2[...]) + lnb2[...]
    coef2 = _dot(us[:, HID:], lvW2d[...]) + lvb2d[...]
    hn = h + t
    h_o[...] = hn

    vc = vc_r[...]
    vce = jnp.concatenate([vc, vch_r[0]], axis=0)
    vcn = vc + coef2 * (vce[1:BE + 1] + vce[2:BE + 2])
    v_o[...] = vcn
    if not final:
        hh_o, vch_o = rest
        hh_o[...] = hn[:2][None]
        vch_o[...] = vcn[:2][None]


def _pick(N, cands):
    for c in cands:
        if N % c == 0 and c <= N:
            return c
    return N


def kernel(positions, features, es_W1, es_b1, es_W2, es_b2,
           ev_W1, ev_b1, ev_W2, ev_b2,
           le_W1, le_b1, le_W2, le_b2,
           ln_W1, ln_b1, ln_W2, ln_b2,
           lv_W1, lv_b1, lv_W2, lv_b2):
    N, F = features.shape
    E = N * K
    NL = le_W1.shape[0]
    HID = es_W1.shape[1]
    SD = es_W2.shape[1]
    VD = ev_W2.shape[1]
    BN = _pick(N, (400, 200, 80, 40, 16, 8))
    nb = N // BN
    BE = BN * K
    BNg = N

    eye = jnp.eye(K, dtype=F32)
    z64 = jnp.zeros((HID,), F32)
    onesV = jnp.ones((VD,), F32)
    zpad = jnp.zeros((128 - 2 * VD,), F32)

    wlen_cat = jnp.concatenate([es_W1[3 * F], ev_W1[3 * F]])
    WlenB = jnp.kron(eye, wlen_cat[None, :]).astype(BF)
    DB2 = jnp.concatenate([
        jnp.kron(eye, jnp.concatenate([onesV, jnp.zeros((VD,), F32),
                                       zpad])[None, :]),
        jnp.kron(eye, jnp.concatenate([jnp.zeros((VD,), F32), onesV,
                                       zpad])[None, :]),
    ], axis=0).astype(BF)

    W1a = jnp.concatenate([es_W1[:F], ev_W1[:F]], axis=1).astype(BF)
    W1b = jnp.concatenate([es_W1[F:2 * F], ev_W1[F:2 * F]], axis=1).astype(BF)
    W1c = jnp.concatenate([es_W1[2 * F:3 * F], ev_W1[2 * F:3 * F]],
                          axis=1).astype(BF)
    b1cat = jnp.concatenate([es_b1, ev_b1])[None, :]
    W2blk = jnp.zeros((2 * HID, SD + 2 * VD), F32)
    W2blk = (W2blk.at[:HID, :SD].set(es_W2)
             .at[HID:, SD:SD + VD].set(ev_W2)
             .at[HID:, SD + VD:].set(ev_W2)).astype(BF)
    b2cat = jnp.concatenate([es_b2, ev_b2, ev_b2])[None, :]

    posT = positions.T
    posx = posT[0:1]
    posy = posT[1:2]

    full = lambda shape: pl.BlockSpec(shape, lambda i: tuple(0 for _ in shape))
    blk = lambda r, c: pl.BlockSpec((r, c), lambda i: (i, 0))
    halo_in = lambda c: pl.BlockSpec((1, 2, c), lambda i: ((i + 1) % nb, 0, 0))
    halo_out = lambda c: pl.BlockSpec((1, 2, c), lambda i: (i, 0, 0))
    edge_out = lambda c: jax.ShapeDtypeStruct((E, c), F32)
    halo_shape = lambda c: jax.ShapeDtypeStruct((nb, 2, c), F32)

    lenT, dirxT, diryT, geoP = pl.pallas_call(
        functools.partial(_geo_body, BNg),
        grid=(1,),
        in_specs=[full((1, N)), full((1, N))],
        out_specs=[full((N, K)), full((N, K)), full((N, K)),
                   full((N, 4 * K))],
        out_shape=[jax.ShapeDtypeStruct((N, K), BF),
                   jax.ShapeDtypeStruct((N, K), BF),
                   jax.ShapeDtypeStruct((N, K), BF),
                   jax.ShapeDtypeStruct((N, 4 * K), BF)],
    )(posx, posy)

    dxyT = jnp.concatenate([dirxT, diryT], axis=1)

    h, vc, hh, vch = pl.pallas_call(
        functools.partial(_stage1_body, F, BN, BE, SD, VD),
        grid=(nb,),
        in_specs=[
            blk(BN, F), pl.BlockSpec((BN, F), lambda i: ((i + 1) % nb, 0)),
            blk(BN, K), blk(BN, 2 * K),
            full(W1a.shape), full(W1b.shape), full(W1c.shape),
            full(WlenB.shape), full(b1cat.shape),
            full(W2blk.shape), full(b2cat.shape), full(DB2.shape),
        ],
        out_specs=[blk(BE, SD), blk(BE, 2 * VD),
                   halo_out(SD), halo_out(2 * VD)],
        out_shape=[edge_out(SD), edge_out(2 * VD),
                   halo_shape(SD), halo_shape(2 * VD)],
    )(features, features, lenT, dxyT, W1a, W1b, W1c, WlenB, b1cat, W2blk,
      b2cat, DB2)

    wspecs = [
        full((SD, HID)), full((SD, HID)), full((1, HID)),
        full((HID, SD)), full((1, SD)), full((4 * K, 2 * HID * K)),
        full((SD, 2 * HID)), full((SD, 2 * HID)), full((1, 2 * HID)),
        full((HID, SD)), full((1, SD)), full((HID, 2 * VD)),
        full((1, 2 * VD)),
    ]
    in_specs = [blk(BE, SD), halo_in(SD), blk(BE, 2 * VD),
                halo_in(2 * VD), blk(BN, 4 * K)] + wspecs
    layer_mid = pl.pallas_call(
        functools.partial(_layer_body, BE, SD, VD, HID, False),
        grid=(nb,),
        in_specs=in_specs,
        out_specs=[blk(BE, SD), blk(BE, 2 * VD),
                   halo_out(SD), halo_out(2 * VD)],
        out_shape=[edge_out(SD), edge_out(2 * VD),
                   halo_shape(SD), halo_shape(2 * VD)],
    )
    layer_fin = pl.pallas_call(
        functools.partial(_layer_body, BE, SD, VD, HID, True),
        grid=(nb,),
        in_specs=in_specs,
        out_specs=[blk(BE, SD), blk(BE, 2 * VD)],
        out_shape=[edge_out(SD), edge_out(2 * VD)],
    )

    zH = jnp.zeros((HID,), F32)
    for l in range(NL):
        A = le_W1[l][:SD].astype(BF)
        Bm = le_W1[l][SD:2 * SD].astype(BF)
        wd = le_W1[l][2 * SD]
        wc = le_W1[l][2 * SD + 1]
        WgeoB = jnp.concatenate([
            jnp.kron(eye, jnp.concatenate([wd, zH])[None, :]),
            jnp.kron(eye, jnp.concatenate([wc, zH])[None, :]),
            jnp.kron(eye, jnp.concatenate([zH, wd])[None, :]),
            jnp.kron(eye, jnp.concatenate([zH, wc])[None, :]),
        ], axis=0).astype(BF)
        W1t = jnp.concatenate([ln_W1[l][:SD], lv_W1[l][:SD]],
                              axis=1).astype(BF)
        W1bot = jnp.concatenate([ln_W1[l][SD:], lv_W1[l][SD:]],
                                axis=1).astype(BF)
        b1u = jnp.concatenate([ln_b1[l], lv_b1[l]])[None, :]
        lvW2d = jnp.concatenate([lv_W2[l], lv_W2[l]], axis=1).astype(BF)
        lvb2d = jnp.concatenate([lv_b2[l], lv_b2[l]])[None, :]
        args = (h, hh, vc, vch, geoP,
                A, Bm, le_b1[l][None, :], le_W2[l].astype(BF),
                (2.0 * le_b2[l])[None, :], WgeoB, W1t, W1bot, b1u,
                ln_W2[l].astype(BF), ln_b2[l][None, :], lvW2d, lvb2d)
        if l < NL - 1:
            h, vc, hh, vch = layer_mid(*args)
        else:
            h, vc = layer_fin(*args)

    v = jnp.stack([vc[:, :VD], vc[:, VD:]], axis=-1)
    return h, v


# speedup vs baseline: 1.0068x; 1.0068x over previous
import functools

import jax
import jax.numpy as jnp
from jax.experimental import pallas as pl
from jax.experimental.pallas import tpu as pltpu

K = 16
K2 = 2
BF = jnp.bfloat16
F32 = jnp.float32


def _dot(a, b):
    return jax.lax.dot_general(a, b, (((1,), (0,)), ((), ())),
                               preferred_element_type=F32)


def _silu(x):
    return x * (0.5 + 0.5 * jnp.tanh(0.5 * x))


def _blockdiag(U):
    R, C = U.shape
    T = jnp.concatenate([U] * K, axis=1)
    r_i = jax.lax.broadcasted_iota(jnp.int32, (R, K * C), 0) % K
    c_i = jax.lax.broadcasted_iota(jnp.int32, (R, K * C), 1) // C
    return jnp.where(r_i == c_i, T, 0.0).astype(BF)


def _geo_body(N, pos_r, lenT_o, dxyT_o, geoP_o):
    pT = jnp.transpose(pos_r[...], (1, 0))
    px = jnp.concatenate([pT[0:1], pT[0:1, :K + 2]], axis=1)
    py = jnp.concatenate([pT[1:2], pT[1:2, :K + 2]], axis=1)
    M = N + 1
    pjx = jnp.concatenate([px[:, k + 1:k + 1 + M] for k in range(K)], axis=0)
    pjy = jnp.concatenate([py[:, k + 1:k + 1 + M] for k in range(K)], axis=0)
    pix = jnp.broadcast_to(px[:, :M], (K, M))
    piy = jnp.broadcast_to(py[:, :M], (K, M))
    relx = pjx - pix
    rely = pjy - piy
    ss = relx * relx + rely * rely
    inv = jax.lax.rsqrt(ss + 1e-12)
    lng = ss * inv
    dirx = relx * inv
    diry = rely * inv
    mpx = 0.5 * (pix + pjx)
    mpy = 0.5 * (piy + pjy)

    def s1(X):
        return jnp.concatenate([X[1:, :N], X[0:1, 1:N + 1]], axis=0)

    def s2(X):
        return jnp.concatenate([X[2:, :N], X[0:2, 1:N + 1]], axis=0)

    dx1 = s1(mpx) - mpx[:, :N]
    dy1 = s1(mpy) - mpy[:, :N]
    dx2 = s2(mpx) - mpx[:, :N]
    dy2 = s2(mpy) - mpy[:, :N]
    d1 = jnp.sqrt(dx1 * dx1 + dy1 * dy1 + 1e-12)
    d2 = jnp.sqrt(dx2 * dx2 + dy2 * dy2 + 1e-12)
    c1 = s1(dirx) * dirx[:, :N] + s1(diry) * diry[:, :N]
    c2 = s2(dirx) * dirx[:, :N] + s2(diry) * diry[:, :N]

    tr = lambda x: jnp.transpose(x, (1, 0)).astype(BF)
    lenT_o[...] = tr(lng[:, :N])
    dxyT_o[...] = jnp.concatenate([tr(dirx[:, :N]), tr(diry[:, :N])], axis=1)
    geoP_o[...] = jnp.concatenate([tr(d1), tr(c1), tr(d2), tr(c2)], axis=1)


def _stage1_body(F, BN, BE, SD, VD, HID,
                 fA, fB, lenT, dxyT, esW1, evW1, b1es, b1ev,
                 esW2, evW2, b2es, b2ev,
                 h_o, vc_o, hh_o, vch_o):
    esW1r = esW1[...]
    evW1r = evW1[...]
    W1a = jnp.concatenate([esW1r[:F], evW1r[:F]], axis=1).astype(BF)
    W1b = jnp.concatenate([esW1r[F:2 * F], evW1r[F:2 * F]], axis=1).astype(BF)
    W1c = jnp.concatenate([esW1r[2 * F:3 * F], evW1r[2 * F:3 * F]],
                          axis=1).astype(BF)
    wlen = jnp.concatenate([esW1r[3 * F:3 * F + 1], evW1r[3 * F:3 * F + 1]],
                           axis=1)
    WlenB = _blockdiag(jnp.broadcast_to(wlen, (K, 2 * HID)))
    db2row = (jax.lax.broadcasted_iota(jnp.int32, (2 * K, 2 * HID), 1)
              // VD == jax.lax.broadcasted_iota(
                  jnp.int32, (2 * K, 2 * HID), 0) // K).astype(F32)
    DB2 = _blockdiag(db2row)
    b1 = jnp.concatenate([b1es[...], b1ev[...]], axis=1)
    zv = jnp.zeros((HID, 2 * VD), F32)
    zs = jnp.zeros((HID, SD), F32)
    W2blk = jnp.concatenate([
        jnp.concatenate([esW2[...], zv], axis=1),
        jnp.concatenate([zs, evW2[...], evW2[...]], axis=1)], axis=0)
    W2blk = W2blk.astype(BF)
    b2 = jnp.concatenate([b2es[...], b2ev[...], b2ev[...]], axis=1)

    fext = jnp.concatenate([fA[...], fB[...]], axis=0).astype(BF)
    fi = jnp.broadcast_to(fext[:BN][:, None, :], (BN, K, F)).reshape(BE, F)
    fj = jnp.concatenate(
        [fext[k + 1:k + 1 + BN][:, None, :] for k in range(K)],
        axis=1).reshape(BE, F)
    diff = jnp.abs(fi - fj)

    lenadd = _dot(lenT[...], WlenB).astype(BF).reshape(BE, 2 * HID)
    pre = (_dot(fi, W1a) + _dot(fj, W1b) + _dot(diff, W1c) + lenadd + b1)
    u = _silu(pre).astype(BF)
    r = _dot(u, W2blk) + b2
    h = r[:, :SD]

    dxy = _dot(dxyT[...], DB2).astype(BF).reshape(BE, 2 * HID)
    vc = r[:, SD:] * dxy[:, :2 * VD]
    h_o[...] = h
    vc_o[...] = vc
    hh_o[...] = h[:2][None]
    vch_o[...] = vc[:2][None]


def _layer_body(BE, SD, VD, HID, l, final,
                h_r, hh_r, vc_r, vch_r, geoP_r,
                leW1_r, leb1_r, leW2_r, leb2_r,
                lnW1_r, lnb1_r, lnW2_r, lnb2_r,
                lvW1_r, lvb1_r, lvW2_r, lvb2_r,
                h_o, v_o, *rest):
    lw1 = leW1_r[l]
    A = lw1[:SD].astype(BF)
    Bm = lw1[SD:2 * SD].astype(BF)
    wd = lw1[2 * SD:2 * SD + 1]
    wc = lw1[2 * SD + 1:2 * SD + 2]
    zh = jnp.zeros((1, HID), F32)
    U = jnp.concatenate([
        jnp.broadcast_to(jnp.concatenate([wd, zh], axis=1), (K, 2 * HID)),
        jnp.broadcast_to(jnp.concatenate([wc, zh], axis=1), (K, 2 * HID)),
        jnp.broadcast_to(jnp.concatenate([zh, wd], axis=1), (K, 2 * HID)),
        jnp.broadcast_to(jnp.concatenate([zh, wc], axis=1), (K, 2 * HID)),
    ], axis=0)
    WgeoB = _blockdiag(U)
    leb1 = leb1_r[l:l + 1]
    leW2 = leW2_r[l].astype(BF)
    leb2x2 = 2.0 * leb2_r[l:l + 1]
    W1t = jnp.concatenate([lnW1_r[l][:SD], lvW1_r[l][:SD]],
                          axis=1).astype(BF)
    W1bot = jnp.concatenate([lnW1_r[l][SD:], lvW1_r[l][SD:]],
                            axis=1).astype(BF)
    b1u = jnp.concatenate([lnb1_r[l:l + 1], lvb1_r[l:l + 1]], axis=1)
    lnW2 = lnW2_r[l].astype(BF)
    lnb2 = lnb2_r[l:l + 1]
    lvW2d = jnp.concatenate([lvW2_r[l], lvW2_r[l]], axis=1).astype(BF)
    lvb2d = jnp.concatenate([lvb2_r[l:l + 1], lvb2_r[l:l + 1]], axis=1)

    h = h_r[...]
    hb = h.astype(BF)
    hextb = jnp.concatenate([hb, hh_r[0].astype(BF)], axis=0)
    hA = _dot(hextb, A)
    s1 = hA[1:BE + 1]
    s2 = hA[2:BE + 2]
    hB = _dot(hb, Bm) + leb1
    g = _dot(geoP_r[...], WgeoB).astype(BF).reshape(BE, 2 * HID)
    m1 = _silu(s1 + hB + g[:, :HID]).astype(BF)
    m2 = _silu(s2 + hB + g[:, HID:]).astype(BF)
    agg = _dot(m1, leW2) + _dot(m2, leW2) + leb2x2

    u = _dot(hb, W1t) + _dot(agg.astype(BF), W1bot) + b1u
    us = _silu(u).astype(BF)
    t = _dot(us[:, :HID], lnW2) + lnb2
    coef2 = _dot(us[:, HID:], lvW2d) + lvb2d
    hn = h + t
    h_o[...] = hn

    vc = vc_r[...]
    vce = jnp.concatenate([vc, vch_r[0]], axis=0)
    vcn = vc + coef2 * (vce[1:BE + 1] + vce[2:BE + 2])
    v_o[...] = vcn
    if not final:
        hh_o, vch_o = rest
        hh_o[...] = hn[:2][None]
        vch_o[...] = vcn[:2][None]


def _pick(N, cands):
    for c in cands:
        if N % c == 0 and c <= N:
            return c
    return N


def kernel(positions, features, es_W1, es_b1, es_W2, es_b2,
           ev_W1, ev_b1, ev_W2, ev_b2,
           le_W1, le_b1, le_W2, le_b2,
           ln_W1, ln_b1, ln_W2, ln_b2,
           lv_W1, lv_b1, lv_W2, lv_b2):
    N, F = features.shape
    E = N * K
    NL = le_W1.shape[0]
    HID = es_W1.shape[1]
    SD = es_W2.shape[1]
    VD = ev_W2.shape[1]
    BN = _pick(N, (400, 200, 80, 40, 16, 8))
    nb = N // BN
    BE = BN * K

    full = lambda shape: pl.BlockSpec(shape, lambda i: tuple(0 for _ in shape))
    blk = lambda r, c: pl.BlockSpec((r, c), lambda i: (i, 0))
    halo_in = lambda c: pl.BlockSpec((1, 2, c), lambda i: ((i + 1) % nb, 0, 0))
    halo_out = lambda c: pl.BlockSpec((1, 2, c), lambda i: (i, 0, 0))
    edge_out = lambda c: jax.ShapeDtypeStruct((E, c), F32)
    halo_shape = lambda c: jax.ShapeDtypeStruct((nb, 2, c), F32)

    lenT, dxyT, geoP = pl.pallas_call(
        functools.partial(_geo_body, N),
        grid=(1,),
        in_specs=[full((N, 2))],
        out_specs=[full((N, K)), full((N, 2 * K)), full((N, 4 * K))],
        out_shape=[jax.ShapeDtypeStruct((N, K), BF),
                   jax.ShapeDtypeStruct((N, 2 * K), BF),
                   jax.ShapeDtypeStruct((N, 4 * K), BF)],
    )(positions)

    h, vc, hh, vch = pl.pallas_call(
        functools.partial(_stage1_body, F, BN, BE, SD, VD, HID),
        grid=(nb,),
        in_specs=[
            blk(BN, F), pl.BlockSpec((BN, F), lambda i: ((i + 1) % nb, 0)),
            blk(BN, K), blk(BN, 2 * K),
            full(es_W1.shape), full(ev_W1.shape),
            full((1, HID)), full((1, HID)),
            full(es_W2.shape), full(ev_W2.shape),
            full((1, SD)), full((1, VD)),
        ],
        out_specs=[blk(BE, SD), blk(BE, 2 * VD),
                   halo_out(SD), halo_out(2 * VD)],
        out_shape=[edge_out(SD), edge_out(2 * VD),
                   halo_shape(SD), halo_shape(2 * VD)],
    )(features, features, lenT, dxyT, es_W1, ev_W1, es_b1[None, :],
      ev_b1[None, :], es_W2, ev_W2, es_b2[None, :], ev_b2[None, :])

    wspecs = [
        full(le_W1.shape), full(le_b1.shape), full(le_W2.shape),
        full(le_b2.shape),
        full(ln_W1.shape), full(ln_b1.shape), full(ln_W2.shape),
        full(ln_b2.shape),
        full(lv_W1.shape), full(lv_b1.shape), full(lv_W2.shape),
        full(lv_b2.shape),
    ]
    in_specs = [blk(BE, SD), halo_in(SD), blk(BE, 2 * VD),
                halo_in(2 * VD), blk(BN, 4 * K)] + wspecs
    wargs = (le_W1, le_b1, le_W2, le_b2, ln_W1, ln_b1, ln_W2, ln_b2,
             lv_W1, lv_b1, lv_W2, lv_b2)

    for l in range(NL):
        final = l == NL - 1
        outs = ([blk(BE, SD), blk(BE, 2 * VD)] if final else
                [blk(BE, SD), blk(BE, 2 * VD), halo_out(SD),
                 halo_out(2 * VD)])
        shapes = ([edge_out(SD), edge_out(2 * VD)] if final else
                  [edge_out(SD), edge_out(2 * VD), halo_shape(SD),
                   halo_shape(2 * VD)])
        res = pl.pallas_call(
            functools.partial(_layer_body, BE, SD, VD, HID, l, final),
            grid=(nb,),
            in_specs=in_specs,
            out_specs=outs,
            out_shape=shapes,
        )(h, hh, vc, vch, geoP, *wargs)
        if final:
            h, vc = res
        else:
            h, vc, hh, vch = res

    v = jnp.stack([vc[:, :VD], vc[:, VD:]], axis=-1)
    return h, v


# speedup vs baseline: 1.2072x; 1.1990x over previous
import functools

import jax
import jax.numpy as jnp
from jax.experimental import pallas as pl
from jax.experimental.pallas import tpu as pltpu

K = 16
K2 = 2
BF = jnp.bfloat16
F32 = jnp.float32


def _dot(a, b):
    return jax.lax.dot_general(a, b, (((1,), (0,)), ((), ())),
                               preferred_element_type=F32)


def _silu(x):
    return x * (0.5 + 0.5 * jnp.tanh(0.5 * x))


def _blockdiag(U):
    R, C = U.shape
    T = jnp.concatenate([U] * K, axis=1)
    r_i = jax.lax.broadcasted_iota(jnp.int32, (R, K * C), 0) % K
    c_i = jax.lax.broadcasted_iota(jnp.int32, (R, K * C), 1) // C
    return jnp.where(r_i == c_i, T, 0.0).astype(BF)


def _geo_body(N, pos_r, lenT_o, dxyT_o, geoP_o):
    pT = jnp.transpose(pos_r[...], (1, 0))
    px = jnp.concatenate([pT[0:1], pT[0:1, :K + 2]], axis=1)
    py = jnp.concatenate([pT[1:2], pT[1:2, :K + 2]], axis=1)
    M = N + 1
    pjx = jnp.concatenate([px[:, k + 1:k + 1 + M] for k in range(K)], axis=0)
    pjy = jnp.concatenate([py[:, k + 1:k + 1 + M] for k in range(K)], axis=0)
    pix = jnp.broadcast_to(px[:, :M], (K, M))
    piy = jnp.broadcast_to(py[:, :M], (K, M))
    relx = pjx - pix
    rely = pjy - piy
    ss = relx * relx + rely * rely
    inv = jax.lax.rsqrt(ss + 1e-12)
    lng = ss * inv
    dirx = relx * inv
    diry = rely * inv
    mpx = 0.5 * (pix + pjx)
    mpy = 0.5 * (piy + pjy)

    def s1(X):
        return jnp.concatenate([X[1:, :N], X[0:1, 1:N + 1]], axis=0)

    def s2(X):
        return jnp.concatenate([X[2:, :N], X[0:2, 1:N + 1]], axis=0)

    dx1 = s1(mpx) - mpx[:, :N]
    dy1 = s1(mpy) - mpy[:, :N]
    dx2 = s2(mpx) - mpx[:, :N]
    dy2 = s2(mpy) - mpy[:, :N]
    d1 = jnp.sqrt(dx1 * dx1 + dy1 * dy1 + 1e-12)
    d2 = jnp.sqrt(dx2 * dx2 + dy2 * dy2 + 1e-12)
    c1 = s1(dirx) * dirx[:, :N] + s1(diry) * diry[:, :N]
    c2 = s2(dirx) * dirx[:, :N] + s2(diry) * diry[:, :N]

    tr = lambda x: jnp.transpose(x, (1, 0)).astype(BF)
    lenT_o[...] = tr(lng[:, :N])
    dxyT_o[...] = jnp.concatenate([tr(dirx[:, :N]), tr(diry[:, :N])], axis=1)
    geoP_o[...] = jnp.concatenate([tr(d1), tr(c1), tr(d2), tr(c2)], axis=1)


def _stage1_body(F, BN, BE, SD, VD, HID,
                 fA, fB, lenT, dxyT, esW1, evW1, b1es, b1ev,
                 esW2, evW2, b2es, b2ev,
                 h_o, vc_o, hh_o, vch_o):
    esW1r = esW1[...]
    evW1r = evW1[...]
    W1a = jnp.concatenate([esW1r[:F], evW1r[:F]], axis=1).astype(BF)
    W1b = jnp.concatenate([esW1r[F:2 * F], evW1r[F:2 * F]], axis=1).astype(BF)
    W1c = jnp.concatenate([esW1r[2 * F:3 * F], evW1r[2 * F:3 * F]],
                          axis=1).astype(BF)
    wlen = jnp.concatenate([esW1r[3 * F:3 * F + 1], evW1r[3 * F:3 * F + 1]],
                           axis=1)
    WlenB = _blockdiag(jnp.broadcast_to(wlen, (K, 2 * HID)))
    db2row = (jax.lax.broadcasted_iota(jnp.int32, (2 * K, 2 * HID), 1)
              // VD == jax.lax.broadcasted_iota(
                  jnp.int32, (2 * K, 2 * HID), 0) // K).astype(F32)
    DB2 = _blockdiag(db2row)
    b1 = jnp.concatenate([b1es[...], b1ev[...]], axis=1)
    zv = jnp.zeros((HID, 2 * VD), F32)
    zs = jnp.zeros((HID, SD), F32)
    W2blk = jnp.concatenate([
        jnp.concatenate([esW2[...], zv], axis=1),
        jnp.concatenate([zs, evW2[...], evW2[...]], axis=1)], axis=0)
    W2blk = W2blk.astype(BF)
    b2 = jnp.concatenate([b2es[...], b2ev[...], b2ev[...]], axis=1)

    fext = jnp.concatenate([fA[...], fB[...]], axis=0).astype(BF)
    fi = jnp.broadcast_to(fext[:BN][:, None, :], (BN, K, F)).reshape(BE, F)
    fj = jnp.concatenate(
        [fext[k + 1:k + 1 + BN][:, None, :] for k in range(K)],
        axis=1).reshape(BE, F)
    diff = jnp.abs(fi - fj)

    lenadd = _dot(lenT[...], WlenB).astype(BF).reshape(BE, 2 * HID)
    pre = (_dot(fi, W1a) + _dot(fj, W1b) + _dot(diff, W1c) + lenadd + b1)
    u = _silu(pre).astype(BF)
    r = _dot(u, W2blk) + b2
    h = r[:, :SD]

    dxy = _dot(dxyT[...], DB2).astype(BF).reshape(BE, 2 * HID)
    vc = r[:, SD:] * dxy[:, :2 * VD]
    h_o[...] = h
    vc_o[...] = vc
    hh_o[...] = h[:2][None]
    vch_o[...] = vc[:2][None]


def _layer_body(BE, SD, VD, HID, l, final,
                h_r, hh_r, vc_r, vch_r, geoP_r,
                leW1_r, leb1_r, leW2_r, leb2_r,
                lnW1_r, lnb1_r, lnW2_r, lnb2_r,
                lvW1_r, lvb1_r, lvW2_r, lvb2_r,
                h_o, v_o, *rest):
    lw1 = leW1_r[l]
    A = lw1[:SD].astype(BF)
    Bm = lw1[SD:2 * SD].astype(BF)
    wd = lw1[2 * SD:2 * SD + 1]
    wc = lw1[2 * SD + 1:2 * SD + 2]
    zh = jnp.zeros((1, HID), F32)
    U = jnp.concatenate([
        jnp.broadcast_to(jnp.concatenate([wd, zh], axis=1), (K, 2 * HID)),
        jnp.broadcast_to(jnp.concatenate([wc, zh], axis=1), (K, 2 * HID)),
        jnp.broadcast_to(jnp.concatenate([zh, wd], axis=1), (K, 2 * HID)),
        jnp.broadcast_to(jnp.concatenate([zh, wc], axis=1), (K, 2 * HID)),
    ], axis=0)
    WgeoB = _blockdiag(U)
    leb1 = leb1_r[l:l + 1]
    leW2 = leW2_r[l].astype(BF)
    leb2x2 = 2.0 * leb2_r[l:l + 1]
    W1t = jnp.concatenate([lnW1_r[l][:SD], lvW1_r[l][:SD]],
                          axis=1).astype(BF)
    W1bot = jnp.concatenate([lnW1_r[l][SD:], lvW1_r[l][SD:]],
                            axis=1).astype(BF)
    b1u = jnp.concatenate([lnb1_r[l:l + 1], lvb1_r[l:l + 1]], axis=1)
    lnW2 = lnW2_r[l].astype(BF)
    lnb2 = lnb2_r[l:l + 1]
    lvW2d = jnp.concatenate([lvW2_r[l], lvW2_r[l]], axis=1).astype(BF)
    lvb2d = jnp.concatenate([lvb2_r[l:l + 1], lvb2_r[l:l + 1]], axis=1)

    h = h_r[...]
    hb = h.astype(BF)
    hextb = jnp.concatenate([hb, hh_r[0].astype(BF)], axis=0)
    hA = _dot(hextb, A)
    s1 = hA[1:BE + 1]
    s2 = hA[2:BE + 2]
    hB = _dot(hb, Bm) + leb1
    g = _dot(geoP_r[...], WgeoB).astype(BF).reshape(BE, 2 * HID)
    m1 = _silu(s1 + hB + g[:, :HID]).astype(BF)
    m2 = _silu(s2 + hB + g[:, HID:]).astype(BF)
    agg = _dot(m1, leW2) + _dot(m2, leW2) + leb2x2

    u = _dot(hb, W1t) + _dot(agg.astype(BF), W1bot) + b1u
    us = _silu(u).astype(BF)
    t = _dot(us[:, :HID], lnW2) + lnb2
    coef2 = _dot(us[:, HID:], lvW2d) + lvb2d
    hn = h + t
    h_o[...] = hn

    vc = vc_r[...]
    vce = jnp.concatenate([vc, vch_r[0]], axis=0)
    vcn = vc + coef2 * (vce[1:BE + 1] + vce[2:BE + 2])
    if final:
        vy_o, = rest
        v_o[...] = vcn[:, :VD]
        vy_o[...] = vcn[:, VD:]
    else:
        v_o[...] = vcn
        hh_o, vch_o = rest
        hh_o[...] = hn[:2][None]
        vch_o[...] = vcn[:2][None]


def _pick(N, cands):
    for c in cands:
        if N % c == 0 and c <= N:
            return c
    return N


def kernel(positions, features, es_W1, es_b1, es_W2, es_b2,
           ev_W1, ev_b1, ev_W2, ev_b2,
           le_W1, le_b1, le_W2, le_b2,
           ln_W1, ln_b1, ln_W2, ln_b2,
           lv_W1, lv_b1, lv_W2, lv_b2):
    N, F = features.shape
    E = N * K
    NL = le_W1.shape[0]
    HID = es_W1.shape[1]
    SD = es_W2.shape[1]
    VD = ev_W2.shape[1]
    BN = _pick(N, (400, 200, 80, 40, 16, 8))
    nb = N // BN
    BE = BN * K

    full = lambda shape: pl.BlockSpec(shape, lambda i: tuple(0 for _ in shape))
    blk = lambda r, c: pl.BlockSpec((r, c), lambda i: (i, 0))
    halo_in = lambda c: pl.BlockSpec((1, 2, c), lambda i: ((i + 1) % nb, 0, 0))
    halo_out = lambda c: pl.BlockSpec((1, 2, c), lambda i: (i, 0, 0))
    edge_out = lambda c: jax.ShapeDtypeStruct((E, c), F32)
    halo_shape = lambda c: jax.ShapeDtypeStruct((nb, 2, c), F32)

    lenT, dxyT, geoP = pl.pallas_call(
        functools.partial(_geo_body, N),
        grid=(1,),
        in_specs=[full((N, 2))],
        out_specs=[full((N, K)), full((N, 2 * K)), full((N, 4 * K))],
        out_shape=[jax.ShapeDtypeStruct((N, K), BF),
                   jax.ShapeDtypeStruct((N, 2 * K), BF),
                   jax.ShapeDtypeStruct((N, 4 * K), BF)],
    )(positions)

    h, vc, hh, vch = pl.pallas_call(
        functools.partial(_stage1_body, F, BN, BE, SD, VD, HID),
        grid=(nb,),
        in_specs=[
            blk(BN, F), pl.BlockSpec((BN, F), lambda i: ((i + 1) % nb, 0)),
            blk(BN, K), blk(BN, 2 * K),
            full(es_W1.shape), full(ev_W1.shape),
            full((1, HID)), full((1, HID)),
            full(es_W2.shape), full(ev_W2.shape),
            full((1, SD)), full((1, VD)),
        ],
        out_specs=[blk(BE, SD), blk(BE, 2 * VD),
                   halo_out(SD), halo_out(2 * VD)],
        out_shape=[edge_out(SD), edge_out(2 * VD),
                   halo_shape(SD), halo_shape(2 * VD)],
    )(features, features, lenT, dxyT, es_W1, ev_W1, es_b1[None, :],
      ev_b1[None, :], es_W2, ev_W2, es_b2[None, :], ev_b2[None, :])

    wspecs = [
        full(le_W1.shape), full(le_b1.shape), full(le_W2.shape),
        full(le_b2.shape),
        full(ln_W1.shape), full(ln_b1.shape), full(ln_W2.shape),
        full(ln_b2.shape),
        full(lv_W1.shape), full(lv_b1.shape), full(lv_W2.shape),
        full(lv_b2.shape),
    ]
    in_specs = [blk(BE, SD), halo_in(SD), blk(BE, 2 * VD),
                halo_in(2 * VD), blk(BN, 4 * K)] + wspecs
    wargs = (le_W1, le_b1, le_W2, le_b2, ln_W1, ln_b1, ln_W2, ln_b2,
             lv_W1, lv_b1, lv_W2, lv_b2)

    for l in range(NL):
        final = l == NL - 1
        outs = ([blk(BE, SD), blk(BE, VD), blk(BE, VD)] if final else
                [blk(BE, SD), blk(BE, 2 * VD), halo_out(SD),
                 halo_out(2 * VD)])
        shapes = ([edge_out(SD), edge_out(VD), edge_out(VD)] if final else
                  [edge_out(SD), edge_out(2 * VD), halo_shape(SD),
                   halo_shape(2 * VD)])
        res = pl.pallas_call(
            functools.partial(_layer_body, BE, SD, VD, HID, l, final),
            grid=(nb,),
            in_specs=in_specs,
            out_specs=outs,
            out_shape=shapes,
        )(h, hh, vc, vch, geoP, *wargs)
        if final:
            h, vxf, vyf = res
        else:
            h, vc, hh, vch = res

    v = jnp.stack([vxf, vyf], axis=-1)
    return h, v
